# baseline (speedup 1.0000x reference)
import functools

import jax
import jax.numpy as jnp
from jax import lax
from jax.experimental import pallas as pl
from jax.experimental.pallas import tpu as pltpu
from jax.experimental.pallas import tpu_sc as plsc

_NC = 2
_NS = 16
_NW = _NC * _NS
_L = 16
_NSUB = 5
_UNROLL = 5


def _sc_segment_partials(energy, idx, num_images, chunk):
    mesh = plsc.VectorSubcoreMesh(core_axis_name="c", subcore_axis_name="s")
    sub = chunk // _NSUB
    nvec = sub // _L
    assert chunk == sub * _NSUB and sub % _L == 0 and nvec % _UNROLL == 0
    pitch = num_images + 1
    acc_len = _L * pitch

    @functools.partial(
        pl.kernel,
        mesh=mesh,
        compiler_params=pltpu.CompilerParams(needs_layout_passes=False),
        out_type=jax.ShapeDtypeStruct((_NW, num_images), jnp.float32),
        scratch_types=[
            pltpu.VMEM((acc_len,), jnp.float32),
            pltpu.VMEM((num_images,), jnp.float32),
            pltpu.VMEM((sub,), jnp.float32),
            pltpu.VMEM((sub,), jnp.float32),
            pltpu.VMEM((sub,), jnp.int32),
            pltpu.VMEM((sub,), jnp.int32),
            pltpu.VMEM((_L,), jnp.int32),
            pltpu.SemaphoreType.DMA,
            pltpu.SemaphoreType.DMA,
        ],
    )
    def k(e_hbm, i_hbm, out_hbm, acc, red, e_v0, e_v1, i_v0, i_v1, tail_v,
          sem0, sem1):
        c = lax.axis_index("c")
        s = lax.axis_index("s")
        wid = c * _NS + s
        base = wid * chunk
        e_bufs = (e_v0, e_v1)
        i_bufs = (i_v0, i_v1)
        sems = (sem0, sem1)

        def start(j):
            b = j % 2
            h = sub // 2
            cs = []
            for src, dst in ((e_hbm, e_bufs[b]), (i_hbm, i_bufs[b])):
                for p in range(2):
                    cs.append(pltpu.async_copy(
                        src.at[pl.ds(base + j * sub + p * h, h)],
                        dst.at[pl.ds(p * h, h)], sems[b]))
            return tuple(cs)

        pend = start(0)
        ctail = pltpu.async_copy(
            i_hbm.at[pl.ds(base + chunk - _L, _L)], tail_v, sems[0])

        zero = jnp.zeros((_L,), jnp.float32)

        @plsc.parallel_loop(0, num_images // _L, unroll=8)
        def _(jv):
            red[pl.ds(jv * _L, _L)] = zero

        ctail.wait()
        for cc in pend:
            cc.wait()
        pend = ()

        lo = lax.reduce_min(i_bufs[0][pl.ds(0, _L)], (0,))
        hi = lax.reduce_max(tail_v[...], (0,))
        loa = lax.shift_left(lax.shift_right_logical(lo, 4), 4)
        nvz = lax.shift_right_logical(hi - loa, 4) + 1

        @plsc.parallel_loop(0, nvz)
        def _(jv):
            for r in range(_L):
                acc[pl.ds(r * pitch + loa + jv * _L, _L)] = zero

        lane_off = lax.iota(jnp.int32, _L) * pitch

        for j in range(_NSUB):
            b = j % 2
            for cc in pend:
                cc.wait()
            if j + 1 < _NSUB:
                pend = start(j + 1)
            else:
                pend = ()
            eb = e_bufs[b]
            ib = i_bufs[b]

            stride = nvec // _UNROLL

            @plsc.parallel_loop(0, stride)
            def _(v):
                for q in range(_UNROLL):
                    sl = pl.ds((q * stride + v) * _L, _L)
                    plsc.addupdate_scatter(
                        acc, [ib[sl] + lane_off], eb[sl])

        @plsc.parallel_loop(0, nvz)
        def _(jv):
            t = acc[pl.ds(loa + jv * _L, _L)]
            for r in range(1, _L):
                t = t + acc[pl.ds(r * pitch + loa + jv * _L, _L)]
            red[pl.ds(loa + jv * _L, _L)] = t

        pltpu.sync_copy(red, out_hbm.at[wid])

    return k(energy, idx)


def _tc_merge(partials, num_images):
    def body(p_ref, o_ref):
        o_ref[...] = jnp.sum(p_ref[...], axis=0)

    return pl.pallas_call(
        body,
        out_shape=jax.ShapeDtypeStruct((num_images,), jnp.float32),
    )(partials)


def kernel(atomic_energy, image_idx, n_atoms):
    n = atomic_energy.shape[0]
    num_images = n_atoms.shape[0]
    assert n % _NW == 0
    chunk = n // _NW

    idx32 = image_idx.astype(jnp.int32)
    energy = atomic_energy.astype(jnp.float32)

    partials = _sc_segment_partials(energy, idx32, num_images, chunk)
    return _tc_merge(partials, num_images)


# speedup vs baseline: 1.0092x; 1.0092x over previous
import functools

import jax
import jax.numpy as jnp
from jax import lax
from jax.experimental import pallas as pl
from jax.experimental.pallas import tpu as pltpu
from jax.experimental.pallas import tpu_sc as plsc

_NC = 2
_NS = 16
_NW = _NC * _NS
_L = 16
_NSUB = 5
_UNROLL = 5


def _sc_segment_partials(energy, idx, num_images, chunk):
    mesh = plsc.VectorSubcoreMesh(core_axis_name="c", subcore_axis_name="s")
    sub = chunk // _NSUB
    nvec = sub // _L
    assert chunk == sub * _NSUB and sub % _L == 0 and nvec % _UNROLL == 0
    pitch = num_images + 1
    acc_len = _L * pitch

    @functools.partial(
        pl.kernel,
        mesh=mesh,
        compiler_params=pltpu.CompilerParams(needs_layout_passes=False),
        out_type=jax.ShapeDtypeStruct((_NW, num_images), jnp.float32),
        scratch_types=[
            pltpu.VMEM((acc_len,), jnp.float32),
            pltpu.VMEM((num_images,), jnp.float32),
            pltpu.VMEM((sub,), jnp.float32),
            pltpu.VMEM((sub,), jnp.float32),
            pltpu.VMEM((sub,), jnp.int32),
            pltpu.VMEM((sub,), jnp.int32),
            pltpu.VMEM((_L,), jnp.int32),
            pltpu.SemaphoreType.DMA,
            pltpu.SemaphoreType.DMA,
        ],
    )
    def k(e_hbm, i_hbm, out_hbm, acc, red, e_v0, e_v1, i_v0, i_v1, tail_v,
          sem0, sem1):
        c = lax.axis_index("c")
        s = lax.axis_index("s")
        wid = c * _NS + s
        base = wid * chunk
        e_bufs = (e_v0, e_v1)
        i_bufs = (i_v0, i_v1)
        sems = (sem0, sem1)

        def start(j):
            b = j % 2
            ce = pltpu.async_copy(
                e_hbm.at[pl.ds(base + j * sub, sub)], e_bufs[b], sems[b])
            ci = pltpu.async_copy(
                i_hbm.at[pl.ds(base + j * sub, sub)], i_bufs[b], sems[b])
            return ce, ci

        pend = start(0)
        ctail = pltpu.async_copy(
            i_hbm.at[pl.ds(base + chunk - _L, _L)], tail_v, sems[0])

        zero = jnp.zeros((_L,), jnp.float32)

        @plsc.parallel_loop(0, num_images // _L, unroll=8)
        def _(jv):
            red[pl.ds(jv * _L, _L)] = zero

        ctail.wait()
        for cc in pend:
            cc.wait()
        pend = ()

        lo = lax.reduce_min(i_bufs[0][pl.ds(0, _L)], (0,))
        hi = lax.reduce_max(tail_v[...], (0,))
        loa = lax.shift_left(lax.shift_right_logical(lo, 4), 4)
        nvz = lax.shift_right_logical(hi - loa, 4) + 1

        @plsc.parallel_loop(0, nvz)
        def _(jv):
            for r in range(_L):
                acc[pl.ds(r * pitch + loa + jv * _L, _L)] = zero

        lane_off = lax.iota(jnp.int32, _L) * pitch

        for j in range(_NSUB):
            b = j % 2
            for cc in pend:
                cc.wait()
            if j + 1 < _NSUB:
                pend = start(j + 1)
            else:
                pend = ()
            eb = e_bufs[b]
            ib = i_bufs[b]

            stride = nvec // _UNROLL

            @plsc.parallel_loop(0, stride)
            def _(v):
                for q in range(_UNROLL):
                    sl = pl.ds((q * stride + v) * _L, _L)
                    plsc.addupdate_scatter(
                        acc, [ib[sl] + lane_off], eb[sl])

        @plsc.parallel_loop(0, nvz)
        def _(jv):
            t = acc[pl.ds(loa + jv * _L, _L)]
            for r in range(1, _L):
                t = t + acc[pl.ds(r * pitch + loa + jv * _L, _L)]
            red[pl.ds(loa + jv * _L, _L)] = t

        pltpu.sync_copy(red, out_hbm.at[wid])

    return k(energy, idx)


def _tc_merge(partials, num_images):
    def body(p_ref, o_ref):
        o_ref[...] = jnp.sum(p_ref[...], axis=0)

    return pl.pallas_call(
        body,
        out_shape=jax.ShapeDtypeStruct((num_images,), jnp.float32),
    )(partials)


def kernel(atomic_energy, image_idx, n_atoms):
    n = atomic_energy.shape[0]
    num_images = n_atoms.shape[0]
    assert n % _NW == 0
    chunk = n // _NW

    idx32 = image_idx.astype(jnp.int32)
    energy = atomic_energy.astype(jnp.float32)

    partials = _sc_segment_partials(energy, idx32, num_images, chunk)
    return _tc_merge(partials, num_images)
